# baseline (speedup 1.0000x reference)
import functools

import jax
import jax.numpy as jnp
from jax import lax
from jax.experimental import pallas as pl
from jax.experimental.pallas import tpu as pltpu
from jax.experimental.pallas import tpu_sc as plsc

_NUM_CORES = 2
_NUM_SUBCORES = 16
_NW = _NUM_CORES * _NUM_SUBCORES

_CH = 128
_NBUF = 4
_NOBUF = 2

_WBLK = 2000


@functools.partial(jax.jit, static_argnames=("n", "d"))
def _gather_rows(xf, table, n, d):
    v = table.shape[0]
    per_w = n // _NW
    ng = per_w // _CH

    @functools.partial(
        pl.pallas_call,
        out_shape=jax.ShapeDtypeStruct((v, 128), jnp.float32),
        grid=(v // _WBLK,),
        in_specs=[pl.BlockSpec((_WBLK, d), lambda i: (i, 0))],
        out_specs=pl.BlockSpec((_WBLK, 128), lambda i: (i, 0)),
    )
    def widen(t_ref, o_ref):
        o_ref[...] = jnp.pad(t_ref[...], ((0, 0), (0, 128 - d)))

    mesh = plsc.VectorSubcoreMesh(core_axis_name="c", subcore_axis_name="s")

    @functools.partial(
        pl.kernel,
        mesh=mesh,
        compiler_params=pltpu.CompilerParams(use_tc_tiling_on_sc=True),
        out_type=jax.ShapeDtypeStruct((n, d), jnp.float32),
        scratch_types=[
            pltpu.VMEM((per_w,), jnp.int32),
        ]
        + [pltpu.VMEM((_CH, 128), jnp.float32)] * _NBUF
        + [pltpu.VMEM((_CH, d), jnp.float32)] * _NOBUF
        + [pltpu.SemaphoreType.DMA] * (_NBUF + _NOBUF),
    )
    def k(x_hbm, wide_hbm, out_hbm, idx_v, *bufs_and_sems):
        rows = bufs_and_sems[:_NBUF]
        outs = bufs_and_sems[_NBUF:_NBUF + _NOBUF]
        gsems = bufs_and_sems[_NBUF + _NOBUF:2 * _NBUF + _NOBUF]
        osems = bufs_and_sems[2 * _NBUF + _NOBUF:]

        wid = lax.axis_index("s") * _NUM_CORES + lax.axis_index("c")
        base = wid * per_w
        pltpu.sync_copy(x_hbm.at[pl.ds(base, per_w)], idx_v)

        def fire(c, slot):
            pltpu.make_async_copy(
                wide_hbm.at[idx_v.at[pl.ds(c * _CH, _CH)]],
                rows[slot],
                gsems[slot],
            ).start()

        def wait_gathers(slot):
            pltpu.make_async_copy(
                wide_hbm.at[idx_v.at[pl.ds(0, _CH)]],
                rows[slot],
                gsems[slot],
            ).wait()

        def extract(slot, oslot):
            def body(i, carry):
                for u in range(4):
                    r = i * 4 + u
                    for q in range(d // 16):
                        outs[oslot][r, pl.ds(q * 16, 16)] = (
                            rows[slot][r, pl.ds(q * 16, 16)]
                        )
                return carry

            lax.fori_loop(0, _CH // 4, body, 0)

        def start_out(c, oslot):
            pltpu.make_async_copy(
                outs[oslot],
                out_hbm.at[pl.ds(base + c * _CH, _CH)],
                osems[oslot],
            ).start()

        def wait_out(oslot):
            pltpu.make_async_copy(
                outs[oslot],
                out_hbm.at[pl.ds(base, _CH)],
                osems[oslot],
            ).wait()

        F = _NBUF - 2
        for c0 in range(F):
            fire(c0, c0)

        def round_body(rnd, carry):
            for b in range(_NBUF):
                c = rnd * _NBUF + b
                oslot = b % _NOBUF

                @pl.when(c + F < ng)
                def _(c=c, b=b):
                    fire(c + F, (b + F) % _NBUF)

                wait_gathers(b)

                @pl.when(c - _NOBUF >= 0)
                def _(oslot=oslot):
                    wait_out(oslot)

                extract(b, oslot)
                start_out(c, oslot)
            return carry

        lax.fori_loop(0, ng // _NBUF, round_body, 0)

        for oslot in range(_NOBUF):
            wait_out(oslot)

    wide = widen(table)
    return k(xf, wide)


def kernel(x, table):
    b, h = x.shape
    v, d = table.shape
    n = b * h
    xf = x.reshape(n).astype(jnp.int32)
    out = _gather_rows(xf, table, n, d)
    return out.reshape(b, h, d)


# speedup vs baseline: 1.0678x; 1.0678x over previous
import functools

import jax
import jax.numpy as jnp
from jax import lax
from jax.experimental import pallas as pl
from jax.experimental.pallas import tpu as pltpu
from jax.experimental.pallas import tpu_sc as plsc

_NUM_CORES = 2
_NUM_SUBCORES = 16
_NW = _NUM_CORES * _NUM_SUBCORES

_CH = 128
_NBUF = 4


@functools.partial(jax.jit, static_argnames=("n", "d"))
def _gather_rows(xf, table, n, d):
    per_w = n // _NW
    ng = per_w // _CH

    mesh = plsc.VectorSubcoreMesh(core_axis_name="c", subcore_axis_name="s")

    @functools.partial(
        pl.kernel,
        mesh=mesh,
        compiler_params=pltpu.CompilerParams(use_tc_tiling_on_sc=False),
        out_type=jax.ShapeDtypeStruct((n, d), jnp.float32),
        scratch_types=[
            pltpu.VMEM((per_w,), jnp.int32),
        ]
        + [pltpu.VMEM((_CH, d), jnp.float32)] * _NBUF
        + [pltpu.SemaphoreType.DMA] * (2 * _NBUF),
    )
    def k(x_hbm, table_hbm, out_hbm, idx_v, *bufs_and_sems):
        rows = bufs_and_sems[:_NBUF]
        gsems = bufs_and_sems[_NBUF:2 * _NBUF]
        osems = bufs_and_sems[2 * _NBUF:]

        wid = lax.axis_index("s") * _NUM_CORES + lax.axis_index("c")
        base = wid * per_w
        pltpu.sync_copy(x_hbm.at[pl.ds(base, per_w)], idx_v)

        def fire(c, slot):
            pltpu.make_async_copy(
                table_hbm.at[idx_v.at[pl.ds(c * _CH, _CH)]],
                rows[slot],
                gsems[slot],
            ).start()

        def wait_gathers(slot):
            pltpu.make_async_copy(
                table_hbm.at[idx_v.at[pl.ds(0, _CH)]],
                rows[slot],
                gsems[slot],
            ).wait()

        def start_out(c, slot):
            pltpu.make_async_copy(
                rows[slot], out_hbm.at[pl.ds(base + c * _CH, _CH)], osems[slot]
            ).start()

        def wait_out(slot):
            pltpu.make_async_copy(
                rows[slot], out_hbm.at[pl.ds(base, _CH)], osems[slot]
            ).wait()

        F = _NBUF - 2
        for c0 in range(F):
            fire(c0, c0)

        def round_body(r, carry):
            for b in range(_NBUF):
                c = r * _NBUF + b
                nxt = (b + F) % _NBUF

                @pl.when(c + F < ng)
                def _(c=c, b=b, nxt=nxt):
                    @pl.when(c - 2 >= 0)
                    def _():
                        wait_out(nxt)

                    fire(c + F, nxt)

                wait_gathers(b)
                start_out(c, b)
            return carry

        lax.fori_loop(0, ng // _NBUF, round_body, 0)

        for slot in range(_NBUF):
            wait_out(slot)

    return k(xf, table)


def kernel(x, table):
    b, h = x.shape
    v, d = table.shape
    n = b * h
    xf = x.reshape(n).astype(jnp.int32)
    out = _gather_rows(xf, table, n, d)
    return out.reshape(b, h, d)
